# Initial kernel scaffold; baseline (speedup 1.0000x reference)
import jax
import jax.numpy as jnp
from jax.experimental import pallas as pl


def kernel(y_hat, product_vectors, gt_indices):
    raise NotImplementedError("write your pallas kernel here")


# speedup vs baseline: 4.0097x; 4.0097x over previous
import functools

import jax
import jax.numpy as jnp
from jax.experimental import pallas as pl
from jax.experimental.pallas import tpu as pltpu

_KBLK = 2048
_SWEEPS_A = 8
_SWEEPS_B = 5
_SWEEPS = _SWEEPS_A + _SWEEPS_B
_SIGN = -2147483648


def _keys_kernel(k_real, yn_ref, pn_ref, out_ref):
    j = pl.program_id(0)
    sim = jax.lax.dot_general(
        yn_ref[...], pn_ref[...], (((1,), (1,)), ((), ())),
        preferred_element_type=jnp.float32)
    u = jax.lax.bitcast_convert_type(sim, jnp.int32)
    ukey = jnp.where(u >= 0, u ^ jnp.int32(_SIGN), ~u)
    col = j * _KBLK + jax.lax.broadcasted_iota(jnp.int32, sim.shape, 1)
    out_ref[...] = jnp.where(col < k_real, ukey, 0)


def _select_kernel(k_real, keys_ref, r_ref, out_ref, p_ref, q_ref, rres_ref,
                   hist_ref):
    s = pl.program_id(0)
    j = pl.program_id(1)
    nb = pl.num_programs(1)

    @pl.when((s == 0) & (j == 0))
    def _init():
        p_ref[...] = jnp.zeros_like(p_ref)
        q_ref[...] = jnp.zeros_like(q_ref)
        rres_ref[...] = r_ref[...]

    @pl.when(j == 0)
    def _zero_hist():
        hist_ref[...] = jnp.zeros_like(hist_ref)

    ukey = keys_ref[...]
    col = j * _KBLK + jax.lax.broadcasted_iota(jnp.int32, ukey.shape, 1)
    in_a = s < _SWEEPS_A
    kb = s - _SWEEPS_A
    shift_a = jnp.clip(32 - 4 * s, 0, 31)
    shift_b = jnp.clip(20 - 4 * kb, 0, 31)
    pfx = p_ref[...]
    qfx = q_ref[...]
    srl = jax.lax.shift_right_logical
    elig_a = (s == 0) | (srl(ukey, shift_a) == pfx)
    elig_b = (ukey == pfx) & ((kb == 0) | (srl(col, shift_b) == qfx))
    elig = (in_a & elig_a) | (jnp.logical_not(in_a) & elig_b)
    nib_shift = jnp.where(in_a, jnp.clip(32 - 4 * s - 4, 0, 31),
                          jnp.clip(20 - 4 * kb - 4, 0, 31))
    nib = srl(jnp.where(in_a, ukey, col), nib_shift) & 15
    for n in range(16):
        cnt = jnp.sum((elig & (nib == n)).astype(jnp.int32), axis=1,
                      keepdims=True)
        hist_ref[:, n:n + 1] += cnt

    @pl.when(j == nb - 1)
    def _update():
        h = hist_ref[...]
        cols = []
        run = jnp.zeros_like(h[:, 0:1])
        for n in range(16):
            run = run + h[:, n:n + 1]
            cols.append(run)
        t = jnp.concatenate(cols, axis=1)
        total = t[:, 15:16]
        s_above = total - t
        rres = rres_ref[...]
        n_a = jnp.sum((s_above > rres).astype(jnp.int32), axis=1,
                      keepdims=True)
        n_b = jnp.sum((t <= rres).astype(jnp.int32), axis=1, keepdims=True)
        nstar = jnp.where(in_a, n_a, n_b)
        lane = jax.lax.broadcasted_iota(jnp.int32, h.shape, 1)
        selm = lane == nstar
        s_at = jnp.sum(jnp.where(selm, s_above, 0), axis=1, keepdims=True)
        l_at = jnp.sum(jnp.where(selm, t - h, 0), axis=1, keepdims=True)
        rres_ref[...] = rres - jnp.where(in_a, s_at, l_at)
        p_new = jnp.where(in_a, pfx * 16 + nstar, pfx)
        q_new = jnp.where(in_a, qfx, qfx * 16 + nstar)
        p_ref[...] = p_new
        q_ref[...] = q_new

        @pl.when(s == _SWEEPS - 1)
        def _final():
            inv = 1.0 / (q_new + 1).astype(jnp.float32)
            out_ref[...] = jnp.mean(inv).reshape(1, 1)


def kernel(y_hat, product_vectors, gt_indices):
    b, d = y_hat.shape
    k = product_vectors.shape[0]
    nblk = (k + _KBLK - 1) // _KBLK
    kpad = nblk * _KBLK
    eps = 1e-8
    yn = y_hat / jnp.maximum(
        jnp.linalg.norm(y_hat, axis=-1, keepdims=True), eps)
    pn = product_vectors / jnp.maximum(
        jnp.linalg.norm(product_vectors, axis=-1, keepdims=True), eps)
    pn = jnp.pad(pn, ((0, kpad - k), (0, 0)))
    r = gt_indices.astype(jnp.int32).reshape(b, 1)

    keys = pl.pallas_call(
        functools.partial(_keys_kernel, k),
        grid=(nblk,),
        in_specs=[
            pl.BlockSpec((b, d), lambda j: (0, 0)),
            pl.BlockSpec((_KBLK, d), lambda j: (j, 0)),
        ],
        out_specs=pl.BlockSpec((b, _KBLK), lambda j: (0, j)),
        out_shape=jax.ShapeDtypeStruct((b, kpad), jnp.int32),
    )(yn, pn)

    out = pl.pallas_call(
        functools.partial(_select_kernel, k),
        grid=(_SWEEPS, nblk),
        in_specs=[
            pl.BlockSpec((b, _KBLK), lambda s, j: (0, j)),
            pl.BlockSpec((b, 1), lambda s, j: (0, 0)),
        ],
        out_specs=pl.BlockSpec((1, 1), lambda s, j: (0, 0)),
        out_shape=jax.ShapeDtypeStruct((1, 1), jnp.float32),
        scratch_shapes=[
            pltpu.VMEM((b, 1), jnp.int32),
            pltpu.VMEM((b, 1), jnp.int32),
            pltpu.VMEM((b, 1), jnp.int32),
            pltpu.VMEM((b, 16), jnp.int32),
        ],
    )(keys, r)
    return out[0, 0]
